# baseline (speedup 1.0000x reference)
import functools

import jax
import jax.numpy as jnp
from jax import lax
from jax.experimental import pallas as pl
from jax.experimental.pallas import tpu as pltpu
from jax.experimental.pallas import tpu_sc as plsc

N_NODES = 10000
N_EDGES = 320000
D = 128

NC = 2
NS = 16
NW = NC * NS
CH = 128
NCHUNK = 80
HC = 40
EPT = NCHUNK * CH
E_PAD = NW * EPT
N_ACC = 10240
RPT = N_ACC // NS


def _prep_body(ei_ref, o_ref):
    pad_n = E_PAD - N_EDGES
    i = lax.broadcasted_iota(jnp.int32, (2, pad_n), 1)
    r = lax.broadcasted_iota(jnp.int32, (2, pad_n), 0)
    pad = jnp.where(r == 0, (i * 37) % N_NODES,
                    N_NODES + i % (N_ACC - N_NODES))
    full = jnp.concatenate([ei_ref[...], pad], axis=1)
    o_ref[...] = full.reshape(2, NW, NCHUNK, CH)


def _prep(edge_index):
    return pl.pallas_call(
        _prep_body,
        in_specs=[pl.BlockSpec((2, N_EDGES), lambda: (0, 0))],
        out_specs=pl.BlockSpec((2, NW, NCHUNK, CH), lambda: (0, 0, 0, 0)),
        out_shape=jax.ShapeDtypeStruct((2, NW, NCHUNK, CH), jnp.int32),
    )(edge_index)


def _finish_body(p0_ref, p1_ref, d0_ref, d1_ref, w_ref, b_ref, o_ref):
    a = p0_ref[0] + p1_ref[0]
    deg = d0_ref[...] + d1_ref[...]
    col = jnp.concatenate(
        [deg[r][:, None] for r in range(deg.shape[0])], axis=0)
    o_ref[...] = lax.dot_general(
        a, w_ref[...], (((1,), (1,)), ((), ())),
        preferred_element_type=jnp.float32,
    ) + col * b_ref[...][None, :]


def _finish(part, deg, W, b):
    blk = 2048
    grid = N_ACC // blk
    off = N_ACC // blk
    return pl.pallas_call(
        _finish_body,
        grid=(grid,),
        in_specs=[
            pl.BlockSpec((1, blk, D), lambda i: (0, i, 0)),
            pl.BlockSpec((1, blk, D), lambda i: (1, i, 0)),
            pl.BlockSpec((blk // D, D), lambda i: (i, 0)),
            pl.BlockSpec((blk // D, D), lambda i: (i + off, 0)),
            pl.BlockSpec((D, D), lambda i: (0, 0)),
            pl.BlockSpec((D,), lambda i: (0,)),
        ],
        out_specs=pl.BlockSpec((blk, D), lambda i: (i, 0)),
        out_shape=jax.ShapeDtypeStruct((N_NODES, D), jnp.float32),
    )(part, part, deg, deg, W, b)


def _sc_body(feat_hbm, idx_hbm, part_hbm, deg_hbm, src_v, dst_v,
             rows_v, rows2_v, ones_v, acc_sh, deg_sh, sem, sem2, semd,
             sems, sems2):
    c = lax.axis_index("c")
    s = lax.axis_index("s")
    wid = c * NS + s

    def _zero_row(r, carry):
        for k in range(D // 16):
            rows_v[r, pl.ds(k * 16, 16)] = jnp.zeros((16,), jnp.float32)
        return carry
    lax.fori_loop(0, CH, _zero_row, 0)
    for k in range(D // 16):
        ones_v[pl.ds(k * 16, 16)] = jnp.ones((16,), jnp.float32)
    for k in range(RPT // CH):
        pltpu.sync_copy(rows_v, acc_sh.at[pl.ds(s * RPT + k * CH, CH)])
        pltpu.sync_copy(rows_v.at[k], deg_sh.at[pl.ds(s * RPT + k * CH, CH)])
    plsc.subcore_barrier()

    def _gather(j, buf, gsem):
        pltpu.async_copy(feat_hbm.at[src_v.at[j]], buf, gsem)

    def _gather_wait(j, buf, gsem):
        pltpu.make_async_copy(feat_hbm.at[src_v.at[j]], buf, gsem).wait()

    def _scatter(j, buf, ssem):
        pltpu.async_copy(buf, acc_sh.at[dst_v.at[j]], ssem, add=True)
        pltpu.async_copy(ones_v, deg_sh.at[dst_v.at[j]], semd, add=True)

    def _scatter_wait(j, buf, ssem):
        pltpu.make_async_copy(buf, acc_sh.at[dst_v.at[j]], ssem).wait()

    for p in range(NCHUNK // HC):
        pltpu.sync_copy(idx_hbm.at[0, wid, pl.ds(p * HC, HC)], src_v)
        pltpu.sync_copy(idx_hbm.at[1, wid, pl.ds(p * HC, HC)], dst_v)
        _gather(0, rows_v, sem)
        _gather_wait(0, rows_v, sem)
        _scatter(0, rows_v, sems)
        _gather(1, rows2_v, sem2)

        def _pair(j, carry):
            a = 2 * j
            _gather_wait(a + 1, rows2_v, sem2)
            _scatter(a + 1, rows2_v, sems2)
            _scatter_wait(a, rows_v, sems)

            @pl.when(j + 1 < HC // 2)
            def _():
                _gather(a + 2, rows_v, sem)
                _gather_wait(a + 2, rows_v, sem)
                _scatter(a + 2, rows_v, sems)
            _scatter_wait(a + 1, rows2_v, sems2)

            @pl.when(j + 1 < HC // 2)
            def _():
                _gather(a + 3, rows2_v, sem2)
            return carry
        lax.fori_loop(0, HC // 2, _pair, 0)

        def _drain(j, carry):
            pltpu.make_async_copy(ones_v, deg_sh.at[dst_v.at[0]],
                                  semd).wait()
            return carry
        lax.fori_loop(0, HC, _drain, 0)
    plsc.subcore_barrier()

    base = s * RPT
    pltpu.sync_copy(acc_sh.at[pl.ds(base, RPT)],
                    part_hbm.at[c, pl.ds(base, RPT)])
    for k in range(RPT // D):
        pltpu.sync_copy(deg_sh.at[pl.ds(base + k * D, D)],
                        deg_hbm.at[c * (N_ACC // D) + s * (RPT // D) + k])


@functools.partial(
    pl.kernel,
    out_type=(
        jax.ShapeDtypeStruct((NC, N_ACC, D), jnp.float32),
        jax.ShapeDtypeStruct((NC * (N_ACC // D), D), jnp.float32),
    ),
    mesh=plsc.VectorSubcoreMesh(
        core_axis_name="c", subcore_axis_name="s", num_cores=NC,
        num_subcores=NS),
    scratch_types=[
        pltpu.VMEM((HC, CH), jnp.int32),
        pltpu.VMEM((HC, CH), jnp.int32),
        pltpu.VMEM((CH, D), jnp.float32),
        pltpu.VMEM((CH, D), jnp.float32),
        pltpu.VMEM((CH,), jnp.float32),
        pltpu.VMEM_SHARED((N_ACC, D), jnp.float32),
        pltpu.VMEM_SHARED((N_ACC,), jnp.float32),
        pltpu.SemaphoreType.DMA,
        pltpu.SemaphoreType.DMA,
        pltpu.SemaphoreType.DMA,
        pltpu.SemaphoreType.DMA,
        pltpu.SemaphoreType.DMA,
    ],
)
def _sc_aggregate(feat_hbm, idx_hbm, part_hbm, deg_hbm, src_v, dst_v,
                  rows_v, rows2_v, ones_v, acc_sh, deg_sh, sem, sem2, semd,
                  sems, sems2):
    _sc_body(feat_hbm, idx_hbm, part_hbm, deg_hbm, src_v, dst_v,
             rows_v, rows2_v, ones_v, acc_sh, deg_sh, sem, sem2, semd,
             sems, sems2)


def kernel(feat, edge_index, W, b):
    idx = _prep(edge_index)
    part, deg = _sc_aggregate(feat, idx)
    return _finish(part, deg, W, b)


# speedup vs baseline: 1.1597x; 1.1597x over previous
import functools

import jax
import jax.numpy as jnp
from jax import lax
from jax.experimental import pallas as pl
from jax.experimental.pallas import tpu as pltpu
from jax.experimental.pallas import tpu_sc as plsc

N_NODES = 10000
N_EDGES = 320000
D = 128

NC = 2
NS = 16
NW = NC * NS
CH = 128
NCHUNK = 80
HC = 40
EPT = NCHUNK * CH
E_PAD = NW * EPT
N_ACC = 10240
RPT = N_ACC // NS


def _prep_body(ei_ref, o_ref):
    pad_n = E_PAD - N_EDGES
    i = lax.broadcasted_iota(jnp.int32, (2, pad_n), 1)
    r = lax.broadcasted_iota(jnp.int32, (2, pad_n), 0)
    pad = jnp.where(r == 0, (i * 37) % N_NODES,
                    N_NODES + i % (N_ACC - N_NODES))
    full = jnp.concatenate([ei_ref[...], pad], axis=1)
    o_ref[...] = full.reshape(2, NW, NCHUNK, CH)


def _prep(edge_index):
    return pl.pallas_call(
        _prep_body,
        in_specs=[pl.BlockSpec((2, N_EDGES), lambda: (0, 0))],
        out_specs=pl.BlockSpec((2, NW, NCHUNK, CH), lambda: (0, 0, 0, 0)),
        out_shape=jax.ShapeDtypeStruct((2, NW, NCHUNK, CH), jnp.int32),
    )(edge_index)


def _finish_body(p0_ref, p1_ref, d0_ref, d1_ref, w_ref, b_ref, o_ref):
    a = p0_ref[0] + p1_ref[0]
    deg = d0_ref[...] + d1_ref[...]
    col = jnp.concatenate(
        [deg[r][:, None] for r in range(deg.shape[0])], axis=0)
    o_ref[...] = lax.dot_general(
        a, w_ref[...], (((1,), (1,)), ((), ())),
        preferred_element_type=jnp.float32,
    ) + col * b_ref[...][None, :]


def _finish(part, deg, W, b):
    blk = 2048
    grid = N_ACC // blk
    off = N_ACC // blk
    return pl.pallas_call(
        _finish_body,
        grid=(grid,),
        in_specs=[
            pl.BlockSpec((1, blk, D), lambda i: (0, i, 0)),
            pl.BlockSpec((1, blk, D), lambda i: (1, i, 0)),
            pl.BlockSpec((blk // D, D), lambda i: (i, 0)),
            pl.BlockSpec((blk // D, D), lambda i: (i + off, 0)),
            pl.BlockSpec((D, D), lambda i: (0, 0)),
            pl.BlockSpec((D,), lambda i: (0,)),
        ],
        out_specs=pl.BlockSpec((blk, D), lambda i: (i, 0)),
        out_shape=jax.ShapeDtypeStruct((N_NODES, D), jnp.float32),
    )(part, part, deg, deg, W, b)


def _sc_body(feat_hbm, idx_hbm, part_hbm, deg_hbm, src_v, dst_v,
             rows_v, rows2_v, ones_v, acc_sh, deg_sh, sem, sem2, semd,
             sems, sems2):
    c = lax.axis_index("c")
    s = lax.axis_index("s")
    wid = c * NS + s

    def _zero_row(r, carry):
        for k in range(D // 16):
            rows_v[r, pl.ds(k * 16, 16)] = jnp.zeros((16,), jnp.float32)
        return carry
    lax.fori_loop(0, CH, _zero_row, 0)
    for k in range(D // 16):
        ones_v[pl.ds(k * 16, 16)] = jnp.ones((16,), jnp.float32)
    for k in range(RPT // CH):
        pltpu.sync_copy(rows_v, acc_sh.at[pl.ds(s * RPT + k * CH, CH)])
        pltpu.sync_copy(rows_v.at[k], deg_sh.at[pl.ds(s * RPT + k * CH, CH)])
    plsc.subcore_barrier()

    def _gather(j, buf, gsem):
        pltpu.async_copy(feat_hbm.at[src_v.at[j]], buf, gsem)

    def _gather_wait(j, buf, gsem):
        pltpu.make_async_copy(feat_hbm.at[src_v.at[j]], buf, gsem).wait()

    for p in range(NCHUNK // HC):
        pltpu.sync_copy(idx_hbm.at[0, wid, pl.ds(p * HC, HC)], src_v)
        pltpu.sync_copy(idx_hbm.at[1, wid, pl.ds(p * HC, HC)], dst_v)
        _gather(0, rows_v, sem)

        def _pair(j, carry):
            a = 2 * j
            _gather(a + 1, rows2_v, sem2)
            pltpu.async_copy(ones_v, deg_sh.at[dst_v.at[a]], semd, add=True)
            _gather_wait(a, rows_v, sem)
            pltpu.sync_copy(rows_v, acc_sh.at[dst_v.at[a]], add=True)

            @pl.when(j + 1 < HC // 2)
            def _():
                _gather(a + 2, rows_v, sem)
            pltpu.async_copy(ones_v, deg_sh.at[dst_v.at[a + 1]], semd,
                             add=True)
            _gather_wait(a + 1, rows2_v, sem2)
            pltpu.sync_copy(rows2_v, acc_sh.at[dst_v.at[a + 1]], add=True)
            return carry
        lax.fori_loop(0, HC // 2, _pair, 0)

        def _drain(j, carry):
            pltpu.make_async_copy(ones_v, deg_sh.at[dst_v.at[0]],
                                  semd).wait()
            return carry
        lax.fori_loop(0, HC, _drain, 0)
    plsc.subcore_barrier()

    base = s * RPT
    pltpu.sync_copy(acc_sh.at[pl.ds(base, RPT)],
                    part_hbm.at[c, pl.ds(base, RPT)])
    for k in range(RPT // D):
        pltpu.sync_copy(deg_sh.at[pl.ds(base + k * D, D)],
                        deg_hbm.at[c * (N_ACC // D) + s * (RPT // D) + k])


@functools.partial(
    pl.kernel,
    out_type=(
        jax.ShapeDtypeStruct((NC, N_ACC, D), jnp.float32),
        jax.ShapeDtypeStruct((NC * (N_ACC // D), D), jnp.float32),
    ),
    mesh=plsc.VectorSubcoreMesh(
        core_axis_name="c", subcore_axis_name="s", num_cores=NC,
        num_subcores=NS),
    scratch_types=[
        pltpu.VMEM((HC, CH), jnp.int32),
        pltpu.VMEM((HC, CH), jnp.int32),
        pltpu.VMEM((CH, D), jnp.float32),
        pltpu.VMEM((CH, D), jnp.float32),
        pltpu.VMEM((CH,), jnp.float32),
        pltpu.VMEM_SHARED((N_ACC, D), jnp.float32),
        pltpu.VMEM_SHARED((N_ACC,), jnp.float32),
        pltpu.SemaphoreType.DMA,
        pltpu.SemaphoreType.DMA,
        pltpu.SemaphoreType.DMA,
        pltpu.SemaphoreType.DMA,
        pltpu.SemaphoreType.DMA,
    ],
)
def _sc_aggregate(feat_hbm, idx_hbm, part_hbm, deg_hbm, src_v, dst_v,
                  rows_v, rows2_v, ones_v, acc_sh, deg_sh, sem, sem2, semd,
                  sems, sems2):
    _sc_body(feat_hbm, idx_hbm, part_hbm, deg_hbm, src_v, dst_v,
             rows_v, rows2_v, ones_v, acc_sh, deg_sh, sem, sem2, semd,
             sems, sems2)


def kernel(feat, edge_index, W, b):
    idx = _prep(edge_index)
    part, deg = _sc_aggregate(feat, idx)
    return _finish(part, deg, W, b)


# speedup vs baseline: 1.1820x; 1.0192x over previous
import functools

import jax
import jax.numpy as jnp
from jax import lax
from jax.experimental import pallas as pl
from jax.experimental.pallas import tpu as pltpu
from jax.experimental.pallas import tpu_sc as plsc

N_NODES = 10000
N_EDGES = 320000
D = 128

NC = 2
NS = 16
NW = NC * NS
CH = 128
NCHUNK = 80
HC = 40
EPT = NCHUNK * CH
E_PAD = NW * EPT
N_ACC = 10240
RPT = N_ACC // NS


def _prep_body(ei_ref, o_ref):
    pad_n = E_PAD - N_EDGES
    i = lax.broadcasted_iota(jnp.int32, (2, pad_n), 1)
    r = lax.broadcasted_iota(jnp.int32, (2, pad_n), 0)
    pad = jnp.where(r == 0, (i * 37) % N_NODES,
                    N_NODES + i % (N_ACC - N_NODES))
    full = jnp.concatenate([ei_ref[...], pad], axis=1)
    o_ref[...] = full.reshape(2, NW, NCHUNK, CH)


def _prep(edge_index):
    return pl.pallas_call(
        _prep_body,
        in_specs=[pl.BlockSpec((2, N_EDGES), lambda: (0, 0))],
        out_specs=pl.BlockSpec((2, NW, NCHUNK, CH), lambda: (0, 0, 0, 0)),
        out_shape=jax.ShapeDtypeStruct((2, NW, NCHUNK, CH), jnp.int32),
    )(edge_index)


def _finish_body(p0_ref, p1_ref, d0_ref, d1_ref, w_ref, b_ref, o_ref):
    a = p0_ref[0] + p1_ref[0]
    deg = d0_ref[...] + d1_ref[...]
    col = jnp.concatenate(
        [deg[r][:, None] for r in range(deg.shape[0])], axis=0)
    o_ref[...] = lax.dot_general(
        a, w_ref[...], (((1,), (1,)), ((), ())),
        preferred_element_type=jnp.float32,
    ) + col * b_ref[...][None, :]


def _finish(part, deg, W, b):
    blk = 2048
    grid = N_ACC // blk
    off = N_ACC // blk
    return pl.pallas_call(
        _finish_body,
        grid=(grid,),
        in_specs=[
            pl.BlockSpec((1, blk, D), lambda i: (0, i, 0)),
            pl.BlockSpec((1, blk, D), lambda i: (1, i, 0)),
            pl.BlockSpec((blk // D, D), lambda i: (i, 0)),
            pl.BlockSpec((blk // D, D), lambda i: (i + off, 0)),
            pl.BlockSpec((D, D), lambda i: (0, 0)),
            pl.BlockSpec((D,), lambda i: (0,)),
        ],
        out_specs=pl.BlockSpec((blk, D), lambda i: (i, 0)),
        out_shape=jax.ShapeDtypeStruct((N_NODES, D), jnp.float32),
    )(part, part, deg, deg, W, b)


def _sc_body(feat_hbm, idx_hbm, part_hbm, deg_hbm, src_v, dst_v,
             rows_v, rows2_v, ones_v, acc_sh, deg_sh, sem, sem2, semd,
             sems, sems2):
    c = lax.axis_index("c")
    s = lax.axis_index("s")
    wid = c * NS + s

    def _zero_row(r, carry):
        for k in range(D // 16):
            rows_v[r, pl.ds(k * 16, 16)] = jnp.zeros((16,), jnp.float32)
        return carry
    lax.fori_loop(0, CH, _zero_row, 0)
    for k in range(D // 16):
        ones_v[pl.ds(k * 16, 16)] = jnp.ones((16,), jnp.float32)
    for k in range(RPT // CH):
        pltpu.sync_copy(rows_v, acc_sh.at[pl.ds(s * RPT + k * CH, CH)])
        pltpu.sync_copy(rows_v.at[k], deg_sh.at[pl.ds(s * RPT + k * CH, CH)])
    plsc.subcore_barrier()

    def _gather(j, buf, gsem):
        pltpu.async_copy(feat_hbm.at[src_v.at[j]], buf, gsem)

    def _gather_wait(j, buf, gsem):
        pltpu.make_async_copy(feat_hbm.at[src_v.at[j]], buf, gsem).wait()

    def _phase(p, pcarry):
        pltpu.sync_copy(idx_hbm.at[0, wid, pl.ds(p * HC, HC)], src_v)
        pltpu.sync_copy(idx_hbm.at[1, wid, pl.ds(p * HC, HC)], dst_v)
        _gather(0, rows_v, sem)

        def _pair(j, carry):
            a = 2 * j
            _gather(a + 1, rows2_v, sem2)
            pltpu.async_copy(ones_v, deg_sh.at[dst_v.at[a]], semd, add=True)
            _gather_wait(a, rows_v, sem)
            pltpu.sync_copy(rows_v, acc_sh.at[dst_v.at[a]], add=True)

            @pl.when(j + 1 < HC // 2)
            def _():
                _gather(a + 2, rows_v, sem)
            pltpu.async_copy(ones_v, deg_sh.at[dst_v.at[a + 1]], semd,
                             add=True)
            _gather_wait(a + 1, rows2_v, sem2)
            pltpu.sync_copy(rows2_v, acc_sh.at[dst_v.at[a + 1]], add=True)
            return carry
        lax.fori_loop(0, HC // 2, _pair, 0)

        def _drain(j, carry):
            pltpu.make_async_copy(ones_v, deg_sh.at[dst_v.at[0]],
                                  semd).wait()
            return carry
        lax.fori_loop(0, HC, _drain, 0)
        return pcarry
    lax.fori_loop(0, NCHUNK // HC, _phase, 0)
    plsc.subcore_barrier()

    base = s * RPT
    for k in range(RPT // D):
        pltpu.async_copy(deg_sh.at[pl.ds(base + k * D, D)],
                         deg_hbm.at[c * (N_ACC // D) + s * (RPT // D) + k],
                         semd)
    pltpu.sync_copy(acc_sh.at[pl.ds(base, RPT)],
                    part_hbm.at[c, pl.ds(base, RPT)])
    for k in range(RPT // D):
        pltpu.make_async_copy(
            deg_sh.at[pl.ds(base + k * D, D)],
            deg_hbm.at[c * (N_ACC // D) + s * (RPT // D) + k], semd).wait()


@functools.partial(
    pl.kernel,
    out_type=(
        jax.ShapeDtypeStruct((NC, N_ACC, D), jnp.float32),
        jax.ShapeDtypeStruct((NC * (N_ACC // D), D), jnp.float32),
    ),
    mesh=plsc.VectorSubcoreMesh(
        core_axis_name="c", subcore_axis_name="s", num_cores=NC,
        num_subcores=NS),
    scratch_types=[
        pltpu.VMEM((HC, CH), jnp.int32),
        pltpu.VMEM((HC, CH), jnp.int32),
        pltpu.VMEM((CH, D), jnp.float32),
        pltpu.VMEM((CH, D), jnp.float32),
        pltpu.VMEM((CH,), jnp.float32),
        pltpu.VMEM_SHARED((N_ACC, D), jnp.float32),
        pltpu.VMEM_SHARED((N_ACC,), jnp.float32),
        pltpu.SemaphoreType.DMA,
        pltpu.SemaphoreType.DMA,
        pltpu.SemaphoreType.DMA,
        pltpu.SemaphoreType.DMA,
        pltpu.SemaphoreType.DMA,
    ],
)
def _sc_aggregate(feat_hbm, idx_hbm, part_hbm, deg_hbm, src_v, dst_v,
                  rows_v, rows2_v, ones_v, acc_sh, deg_sh, sem, sem2, semd,
                  sems, sems2):
    _sc_body(feat_hbm, idx_hbm, part_hbm, deg_hbm, src_v, dst_v,
             rows_v, rows2_v, ones_v, acc_sh, deg_sh, sem, sem2, semd,
             sems, sems2)


def kernel(feat, edge_index, W, b):
    idx = _prep(edge_index)
    part, deg = _sc_aggregate(feat, idx)
    return _finish(part, deg, W, b)
